# baseline (speedup 1.0000x reference)
import functools

import jax
import jax.numpy as jnp
from jax import lax
from jax.experimental import pallas as pl
from jax.experimental.pallas import tpu as pltpu
from jax.experimental.pallas import tpu_sc as plsc

N = 50000
E = 1600000
NUM_GRAPHS = 128

NC = 2
NS = 16
NW = NC * NS
L = 16

MASK_HI = jnp.int32(-65536)
MASK_LO = jnp.int32(0xFFFF)


def _norm_newton(d2):
    xi = lax.bitcast_convert_type(d2, jnp.int32)
    yi = jnp.int32(0x5F3759DF) - (xi >> 1)
    y = lax.bitcast_convert_type(yi, jnp.float32)
    y = y * (1.5 - 0.5 * d2 * y * y)
    y = y * (1.5 - 0.5 * d2 * y * y)
    return d2 * y


def _unpack_hi(v):
    return lax.bitcast_convert_type(v & MASK_HI, jnp.float32)


def _unpack_lo_f(v):
    return lax.bitcast_convert_type(v << 16, jnp.float32)


def _make_edge_kernel(n_edges, chunk, n_chunks):
    per_w = n_edges // NW
    assert per_w == chunk * n_chunks and n_chunks % 2 == 1
    npair = (n_chunks - 1) // 2
    groups = chunk // L
    c2 = 2 * chunk

    @functools.partial(
        pl.kernel,
        mesh=plsc.VectorSubcoreMesh(core_axis_name="c", subcore_axis_name="s"),
        compiler_params=pltpu.CompilerParams(needs_layout_passes=False),
        out_type=[
            jax.ShapeDtypeStruct((n_edges,), jnp.float32),
            jax.ShapeDtypeStruct((n_edges,), jnp.float32),
            jax.ShapeDtypeStruct((n_edges,), jnp.float32),
            jax.ShapeDtypeStruct((n_edges,), jnp.float32),
            jax.ShapeDtypeStruct((NW, NUM_GRAPHS), jnp.int32),
        ],
        scratch_types=[
            pltpu.VMEM_SHARED((N,), jnp.int32),
            pltpu.VMEM_SHARED((N,), jnp.int32),
            pltpu.VMEM((c2,), jnp.int32), pltpu.VMEM((c2,), jnp.int32),
            pltpu.VMEM((c2,), jnp.int32), pltpu.VMEM((c2,), jnp.int32),
            pltpu.VMEM((c2,), jnp.int32), pltpu.VMEM((c2,), jnp.int32),
            pltpu.VMEM((chunk,), jnp.float32), pltpu.VMEM((chunk,), jnp.float32),
            pltpu.VMEM((chunk,), jnp.float32), pltpu.VMEM((chunk,), jnp.float32),
            pltpu.VMEM((chunk,), jnp.float32), pltpu.VMEM((chunk,), jnp.float32),
            pltpu.VMEM((chunk,), jnp.float32), pltpu.VMEM((chunk,), jnp.float32),
            pltpu.VMEM((NUM_GRAPHS,), jnp.int32),
            pltpu.VMEM((NUM_GRAPHS,), jnp.int32),
            pltpu.VMEM((NUM_GRAPHS,), jnp.int32),
            pltpu.VMEM((NUM_GRAPHS,), jnp.int32),
            pltpu.SemaphoreType.DMA, pltpu.SemaphoreType.DMA,
            pltpu.SemaphoreType.DMA, pltpu.SemaphoreType.DMA,
            pltpu.SemaphoreType.DMA, pltpu.SemaphoreType.DMA,
        ],
    )
    def _edge_kernel(txy_hbm, tzb_hbm, src_hbm, dst_hbm,
                     dvx_out, dvy_out, dvz_out, dist_out, nb_out,
                     txy_sh, tzb_sh,
                     idxA, idxB, exyA, exyB, ezbA, ezbB,
                     oxA, oxB, oyA, oyB, ozA, ozB, odA, odB,
                     hist0, hist1, hist2, hist3,
                     semIA, semIB, semGA, semGB, semOA, semOB):
        cid = lax.axis_index("c")
        sid = lax.axis_index("s")
        wid = sid * NC + cid

        idx = [idxA, idxB]
        exy = [exyA, exyB]
        ezb = [ezbA, ezbB]
        ox = [oxA, oxB]
        oy = [oyA, oyB]
        oz = [ozA, ozB]
        od = [odA, odB]
        semI = [semIA, semIB]
        semG = [semGA, semGB]
        semO = [semOA, semOB]

        @pl.when(sid == 0)
        def _():
            pltpu.sync_copy(txy_hbm, txy_sh)
            pltpu.sync_copy(tzb_hbm, tzb_sh)

        plsc.subcore_barrier()

        hists = [hist0, hist1, hist2, hist3]
        zeros16 = jnp.zeros((L,), jnp.int32)
        for h in hists:
            for k in range(NUM_GRAPHS // L):
                h[pl.ds(k * L, L)] = zeros16

        ones_i = jnp.full((L,), 1, jnp.int32)

        def start_idx(k, p):
            base = wid * per_w + k * chunk
            pltpu.async_copy(src_hbm.at[pl.ds(base, chunk)],
                             idx[p].at[pl.ds(0, chunk)], semI[p])
            pltpu.async_copy(dst_hbm.at[pl.ds(base, chunk)],
                             idx[p].at[pl.ds(chunk, chunk)], semI[p])

        def wait_idx(p):
            pltpu.make_async_copy(src_hbm.at[pl.ds(0, chunk)],
                                  idx[p].at[pl.ds(0, chunk)], semI[p]).wait()
            pltpu.make_async_copy(dst_hbm.at[pl.ds(0, chunk)],
                                  idx[p].at[pl.ds(chunk, chunk)], semI[p]).wait()

        def start_gathers(p):
            pltpu.async_copy(txy_sh.at[idx[p]], exy[p], semG[p])
            pltpu.async_copy(tzb_sh.at[idx[p]], ezb[p], semG[p])

        def wait_gathers(p):
            pltpu.make_async_copy(txy_sh.at[idx[p]], exy[p], semG[p]).wait()
            pltpu.make_async_copy(tzb_sh.at[idx[p]], ezb[p], semG[p]).wait()

        def start_out(k, p):
            base = wid * per_w + k * chunk
            pltpu.async_copy(ox[p], dvx_out.at[pl.ds(base, chunk)], semO[p])
            pltpu.async_copy(oy[p], dvy_out.at[pl.ds(base, chunk)], semO[p])
            pltpu.async_copy(oz[p], dvz_out.at[pl.ds(base, chunk)], semO[p])
            pltpu.async_copy(od[p], dist_out.at[pl.ds(base, chunk)], semO[p])

        def wait_out(p):
            pltpu.make_async_copy(ox[p], dvx_out.at[pl.ds(0, chunk)], semO[p]).wait()
            pltpu.make_async_copy(oy[p], dvy_out.at[pl.ds(0, chunk)], semO[p]).wait()
            pltpu.make_async_copy(oz[p], dvz_out.at[pl.ds(0, chunk)], semO[p]).wait()
            pltpu.make_async_copy(od[p], dist_out.at[pl.ds(0, chunk)], semO[p]).wait()

        def compute(p):
            @plsc.parallel_loop(0, chunk, step=L, unroll=4)
            def _(b):
                vxyj = exy[p][pl.ds(b, L)]
                vzbj = ezb[p][pl.ds(b, L)]
                vxyi = exy[p][pl.ds(chunk + b, L)]
                vzbi = ezb[p][pl.ds(chunk + b, L)]
                dvx = _unpack_hi(vxyj) - _unpack_hi(vxyi)
                dvy = _unpack_lo_f(vxyj) - _unpack_lo_f(vxyi)
                dvz = _unpack_hi(vzbj) - _unpack_hi(vzbi)
                d2 = dvx * dvx + dvy * dvy + dvz * dvz
                ox[p][pl.ds(b, L)] = dvx
                oy[p][pl.ds(b, L)] = dvy
                oz[p][pl.ds(b, L)] = dvz
                od[p][pl.ds(b, L)] = _norm_newton(d2)

            def hist_body(g, _):
                b = 4 * g * L
                for q in range(4):
                    vzbi = ezb[p][pl.ds(chunk + b + q * L, L)]
                    plsc.addupdate_scatter(hists[q], [vzbi & MASK_LO], ones_i)
                return 0

            lax.fori_loop(0, groups // 4, hist_body, 0)
            for q in range(groups % 4):
                b = (groups // 4) * 4 * L + q * L
                vzbi = ezb[p][pl.ds(chunk + b, L)]
                plsc.addupdate_scatter(hists[q], [vzbi & MASK_LO], ones_i)

        start_idx(0, 0)
        wait_idx(0)
        start_gathers(0)
        start_idx(1, 1)

        def pair_body(m, _):
            k = 2 * m
            wait_gathers(0)
            wait_idx(1)
            start_gathers(1)
            start_idx(k + 2, 0)

            @pl.when(m > 0)
            def _():
                wait_out(0)
            compute(0)
            start_out(k, 0)

            wait_gathers(1)

            @pl.when(m < npair - 1)
            def _():
                start_idx(k + 3, 1)
            wait_idx(0)
            start_gathers(0)

            @pl.when(m > 0)
            def _():
                wait_out(1)
            compute(1)
            start_out(k + 1, 1)
            return 0

        lax.fori_loop(0, npair, pair_body, 0)

        wait_gathers(0)
        wait_out(0)
        compute(0)
        start_out(n_chunks - 1, 0)
        wait_out(0)
        wait_out(1)

        for k in range(NUM_GRAPHS // L):
            s = pl.ds(k * L, L)
            hist0[s] = hist0[s] + hist1[s] + hist2[s] + hist3[s]
        pltpu.sync_copy(hist0, nb_out.at[wid])

    return _edge_kernel


EA = 960000
EB = E - EA
_kernel_a = _make_edge_kernel(EA, 2000, 15)
_kernel_b = _make_edge_kernel(EB, 4000, 5)

_SPLIT_B = 25600


def _split_body(ei_ref, src_ref, dst_ref):
    src_ref[...] = ei_ref[0, :]
    dst_ref[...] = ei_ref[1, :]


def _split_edge_index(edge_index):
    return pl.pallas_call(
        _split_body,
        grid=(pl.cdiv(E, _SPLIT_B),),
        in_specs=[pl.BlockSpec((2, _SPLIT_B), lambda i: (0, i))],
        out_specs=[pl.BlockSpec((_SPLIT_B,), lambda i: (i,)),
                   pl.BlockSpec((_SPLIT_B,), lambda i: (i,))],
        out_shape=[jax.ShapeDtypeStruct((E,), jnp.int32),
                   jax.ShapeDtypeStruct((E,), jnp.int32)],
    )(edge_index)


def kernel(pos, edge_index, batch):
    xu = lax.bitcast_convert_type(
        pos[:, 0].astype(jnp.bfloat16), jnp.uint16).astype(jnp.uint32)
    yu = lax.bitcast_convert_type(
        pos[:, 1].astype(jnp.bfloat16), jnp.uint16).astype(jnp.uint32)
    zu = lax.bitcast_convert_type(
        pos[:, 2].astype(jnp.bfloat16), jnp.uint16).astype(jnp.uint32)
    txy = lax.bitcast_convert_type((xu << 16) | yu, jnp.int32)
    tzb = lax.bitcast_convert_type(
        (zu << 16) | batch.astype(jnp.uint32), jnp.int32)
    src, dst = _split_edge_index(edge_index)
    ax, ay, az, ad, anb = _kernel_a(txy, tzb, src[:EA], dst[:EA])
    bx, by, bz, bd, bnb = _kernel_b(txy, tzb, src[EA:], dst[EA:])
    dva = jnp.stack([ax, ay, az], axis=1)
    dvb = jnp.stack([bx, by, bz], axis=1)
    distance_vec = jnp.concatenate([dva, dvb], axis=0)
    edge_dist = jnp.concatenate([ad, bd], axis=0)
    neighbors = jnp.sum(anb, axis=0, dtype=jnp.int32) + \
        jnp.sum(bnb, axis=0, dtype=jnp.int32)
    n_edges = edge_index.shape[1]
    cell_offsets = jnp.zeros((n_edges, 3), dtype=jnp.float32)
    cell_offset_distances = jnp.zeros((n_edges, 3), dtype=jnp.float32)
    return (edge_index, edge_dist, distance_vec, cell_offsets,
            cell_offset_distances, neighbors)


# speedup vs baseline: 1.3342x; 1.3342x over previous
import functools

import jax
import jax.numpy as jnp
from jax import lax
from jax.experimental import pallas as pl
from jax.experimental.pallas import tpu as pltpu
from jax.experimental.pallas import tpu_sc as plsc

N = 50000
E = 1600000
NUM_GRAPHS = 128

NC = 2
NS = 16
NW = NC * NS
L = 16

MASK_HI = jnp.int32(-65536)
MASK_LO = jnp.int32(0xFFFF)


def _norm_newton(d2):
    xi = lax.bitcast_convert_type(d2, jnp.int32)
    yi = jnp.int32(0x5F3759DF) - (xi >> 1)
    y = lax.bitcast_convert_type(yi, jnp.float32)
    y = y * (1.5 - 0.5 * d2 * y * y)
    y = y * (1.5 - 0.5 * d2 * y * y)
    return d2 * y


def _unpack_hi(v):
    return lax.bitcast_convert_type(v & MASK_HI, jnp.float32)


def _unpack_lo_f(v):
    return lax.bitcast_convert_type(v << 16, jnp.float32)


def _make_edge_kernel(n_edges, chunk, n_chunks):
    per_w = n_edges // NW
    assert per_w == chunk * n_chunks and n_chunks % 2 == 1
    npair = (n_chunks - 1) // 2
    groups = chunk // L
    c2 = 2 * chunk

    @functools.partial(
        pl.kernel,
        mesh=plsc.VectorSubcoreMesh(core_axis_name="c", subcore_axis_name="s"),
        compiler_params=pltpu.CompilerParams(needs_layout_passes=False),
        out_type=[
            jax.ShapeDtypeStruct((n_edges,), jnp.float32),
            jax.ShapeDtypeStruct((n_edges,), jnp.float32),
            jax.ShapeDtypeStruct((n_edges,), jnp.float32),
            jax.ShapeDtypeStruct((n_edges,), jnp.float32),
            jax.ShapeDtypeStruct((NW, NUM_GRAPHS), jnp.int32),
        ],
        scratch_types=[
            pltpu.VMEM_SHARED((N,), jnp.int32),
            pltpu.VMEM_SHARED((N,), jnp.int32),
            pltpu.VMEM((c2,), jnp.int32), pltpu.VMEM((c2,), jnp.int32),
            pltpu.VMEM((c2,), jnp.int32), pltpu.VMEM((c2,), jnp.int32),
            pltpu.VMEM((c2,), jnp.int32), pltpu.VMEM((c2,), jnp.int32),
            pltpu.VMEM((chunk,), jnp.float32), pltpu.VMEM((chunk,), jnp.float32),
            pltpu.VMEM((chunk,), jnp.float32), pltpu.VMEM((chunk,), jnp.float32),
            pltpu.VMEM((chunk,), jnp.float32), pltpu.VMEM((chunk,), jnp.float32),
            pltpu.VMEM((chunk,), jnp.float32), pltpu.VMEM((chunk,), jnp.float32),
            pltpu.VMEM((NUM_GRAPHS,), jnp.int32),
            pltpu.VMEM((NUM_GRAPHS,), jnp.int32),
            pltpu.VMEM((NUM_GRAPHS,), jnp.int32),
            pltpu.VMEM((NUM_GRAPHS,), jnp.int32),
            pltpu.SemaphoreType.DMA, pltpu.SemaphoreType.DMA,
            pltpu.SemaphoreType.DMA, pltpu.SemaphoreType.DMA,
            pltpu.SemaphoreType.DMA, pltpu.SemaphoreType.DMA,
        ],
    )
    def _edge_kernel(txy_hbm, tzb_hbm, src_hbm, dst_hbm,
                     dvx_out, dvy_out, dvz_out, dist_out, nb_out,
                     txy_sh, tzb_sh,
                     idxA, idxB, exyA, exyB, ezbA, ezbB,
                     oxA, oxB, oyA, oyB, ozA, ozB, odA, odB,
                     hist0, hist1, hist2, hist3,
                     semIA, semIB, semGA, semGB, semOA, semOB):
        cid = lax.axis_index("c")
        sid = lax.axis_index("s")
        wid = sid * NC + cid

        idx = [idxA, idxB]
        exy = [exyA, exyB]
        ezb = [ezbA, ezbB]
        ox = [oxA, oxB]
        oy = [oyA, oyB]
        oz = [ozA, ozB]
        od = [odA, odB]
        semI = [semIA, semIB]
        semG = [semGA, semGB]
        semO = [semOA, semOB]

        @pl.when(sid == 0)
        def _():
            pltpu.sync_copy(txy_hbm, txy_sh)
            pltpu.sync_copy(tzb_hbm, tzb_sh)

        plsc.subcore_barrier()

        hists = [hist0, hist1, hist2, hist3]
        zeros16 = jnp.zeros((L,), jnp.int32)
        for h in hists:
            for k in range(NUM_GRAPHS // L):
                h[pl.ds(k * L, L)] = zeros16

        ones_i = jnp.full((L,), 1, jnp.int32)

        def start_idx(k, p):
            base = wid * per_w + k * chunk
            pltpu.async_copy(src_hbm.at[pl.ds(base, chunk)],
                             idx[p].at[pl.ds(0, chunk)], semI[p])
            pltpu.async_copy(dst_hbm.at[pl.ds(base, chunk)],
                             idx[p].at[pl.ds(chunk, chunk)], semI[p])

        def wait_idx(p):
            pltpu.make_async_copy(src_hbm.at[pl.ds(0, chunk)],
                                  idx[p].at[pl.ds(0, chunk)], semI[p]).wait()
            pltpu.make_async_copy(dst_hbm.at[pl.ds(0, chunk)],
                                  idx[p].at[pl.ds(chunk, chunk)], semI[p]).wait()

        def start_gathers(p):
            pltpu.async_copy(txy_sh.at[idx[p]], exy[p], semG[p])
            pltpu.async_copy(tzb_sh.at[idx[p]], ezb[p], semG[p])

        def wait_gathers(p):
            pltpu.make_async_copy(txy_sh.at[idx[p]], exy[p], semG[p]).wait()
            pltpu.make_async_copy(tzb_sh.at[idx[p]], ezb[p], semG[p]).wait()

        def start_out(k, p):
            base = wid * per_w + k * chunk
            pltpu.async_copy(ox[p], dvx_out.at[pl.ds(base, chunk)], semO[p])
            pltpu.async_copy(oy[p], dvy_out.at[pl.ds(base, chunk)], semO[p])
            pltpu.async_copy(oz[p], dvz_out.at[pl.ds(base, chunk)], semO[p])
            pltpu.async_copy(od[p], dist_out.at[pl.ds(base, chunk)], semO[p])

        def wait_out(p):
            pltpu.make_async_copy(ox[p], dvx_out.at[pl.ds(0, chunk)], semO[p]).wait()
            pltpu.make_async_copy(oy[p], dvy_out.at[pl.ds(0, chunk)], semO[p]).wait()
            pltpu.make_async_copy(oz[p], dvz_out.at[pl.ds(0, chunk)], semO[p]).wait()
            pltpu.make_async_copy(od[p], dist_out.at[pl.ds(0, chunk)], semO[p]).wait()

        def compute(p):
            @plsc.parallel_loop(0, chunk, step=L, unroll=4)
            def _(b):
                vxyj = exy[p][pl.ds(b, L)]
                vzbj = ezb[p][pl.ds(b, L)]
                vxyi = exy[p][pl.ds(chunk + b, L)]
                vzbi = ezb[p][pl.ds(chunk + b, L)]
                dvx = _unpack_hi(vxyj) - _unpack_hi(vxyi)
                dvy = _unpack_lo_f(vxyj) - _unpack_lo_f(vxyi)
                dvz = _unpack_hi(vzbj) - _unpack_hi(vzbi)
                d2 = dvx * dvx + dvy * dvy + dvz * dvz
                ox[p][pl.ds(b, L)] = dvx
                oy[p][pl.ds(b, L)] = dvy
                oz[p][pl.ds(b, L)] = dvz
                od[p][pl.ds(b, L)] = _norm_newton(d2)

            def hist_body(g, _):
                b = 4 * g * L
                for q in range(4):
                    vzbi = ezb[p][pl.ds(chunk + b + q * L, L)]
                    plsc.addupdate_scatter(hists[q], [vzbi & MASK_LO], ones_i)
                return 0

            lax.fori_loop(0, groups // 4, hist_body, 0)
            for q in range(groups % 4):
                b = (groups // 4) * 4 * L + q * L
                vzbi = ezb[p][pl.ds(chunk + b, L)]
                plsc.addupdate_scatter(hists[q], [vzbi & MASK_LO], ones_i)

        start_idx(0, 0)
        wait_idx(0)
        start_gathers(0)
        start_idx(1, 1)

        def pair_body(m, _):
            k = 2 * m
            wait_gathers(0)
            wait_idx(1)
            start_gathers(1)
            start_idx(k + 2, 0)

            @pl.when(m > 0)
            def _():
                wait_out(0)
            compute(0)
            start_out(k, 0)

            wait_gathers(1)

            @pl.when(m < npair - 1)
            def _():
                start_idx(k + 3, 1)
            wait_idx(0)
            start_gathers(0)

            @pl.when(m > 0)
            def _():
                wait_out(1)
            compute(1)
            start_out(k + 1, 1)
            return 0

        lax.fori_loop(0, npair, pair_body, 0)

        wait_gathers(0)
        wait_out(0)
        compute(0)
        start_out(n_chunks - 1, 0)
        wait_out(0)
        wait_out(1)

        for k in range(NUM_GRAPHS // L):
            s = pl.ds(k * L, L)
            hist0[s] = hist0[s] + hist1[s] + hist2[s] + hist3[s]
        pltpu.sync_copy(hist0, nb_out.at[wid])

    return _edge_kernel


EA = 960000
EB = E - EA
_kernel_a = _make_edge_kernel(EA, 2000, 15)
_kernel_b = _make_edge_kernel(EB, 4000, 5)


def kernel(pos, edge_index, batch):
    n_edges = edge_index.shape[1]
    cell_offsets = jnp.zeros((n_edges, 3), dtype=jnp.float32)
    cell_offset_distances = jnp.zeros((n_edges, 3), dtype=jnp.float32)
    xu = lax.bitcast_convert_type(
        pos[:, 0].astype(jnp.bfloat16), jnp.uint16).astype(jnp.uint32)
    yu = lax.bitcast_convert_type(
        pos[:, 1].astype(jnp.bfloat16), jnp.uint16).astype(jnp.uint32)
    zu = lax.bitcast_convert_type(
        pos[:, 2].astype(jnp.bfloat16), jnp.uint16).astype(jnp.uint32)
    txy = lax.bitcast_convert_type((xu << 16) | yu, jnp.int32)
    tzb = lax.bitcast_convert_type(
        (zu << 16) | batch.astype(jnp.uint32), jnp.int32)
    bx, by, bz, bd, bnb = _kernel_b(txy, tzb,
                                    edge_index[0, EA:], edge_index[1, EA:])
    ax, ay, az, ad, anb = _kernel_a(txy, tzb,
                                    edge_index[0, :EA], edge_index[1, :EA])
    e0 = jnp.array([1.0, 0.0, 0.0], jnp.float32)
    e1 = jnp.array([0.0, 1.0, 0.0], jnp.float32)
    e2 = jnp.array([0.0, 0.0, 1.0], jnp.float32)
    a3 = ax[:, None] * e0 + ay[:, None] * e1 + az[:, None] * e2
    b3 = bx[:, None] * e0 + by[:, None] * e1 + bz[:, None] * e2
    dv = jnp.zeros((E, 3), jnp.float32)
    dv = lax.dynamic_update_slice(dv, a3, (0, 0))
    distance_vec = lax.dynamic_update_slice(dv, b3, (EA, 0))
    edge_dist = jnp.concatenate([ad, bd], axis=0)
    neighbors = jnp.sum(anb, axis=0, dtype=jnp.int32) + \
        jnp.sum(bnb, axis=0, dtype=jnp.int32)
    return (edge_index, edge_dist, distance_vec, cell_offsets,
            cell_offset_distances, neighbors)
